# Initial kernel scaffold; baseline (speedup 1.0000x reference)
import jax
import jax.numpy as jnp
from jax.experimental import pallas as pl


def kernel(x, W1, b1, W2, b2, Wg, bg):
    raise NotImplementedError("write your pallas kernel here")


# speedup vs baseline: 5.5683x; 5.5683x over previous
import functools

import jax
import jax.numpy as jnp
from jax.experimental import pallas as pl
from jax.experimental.pallas import tpu as pltpu

E = 64
TOPK = 2
T = 2048
D = 768
DFF = 4 * D
P = T * TOPK
TILE = 128
NT = P // TILE + E - 1
NT = 96
R = NT * TILE


def _gate_kernel(x_ref, wg_ref, bg_ref, gs_ref, ti_ref, tv_ref):
    g = jax.lax.dot_general(x_ref[...], wg_ref[...],
                            (((1,), (1,)), ((), ())),
                            preferred_element_type=jnp.float32)
    g = jax.nn.sigmoid(g + bg_ref[...])
    gs_ref[...] = g
    col = jax.lax.broadcasted_iota(jnp.int32, (T, E), 1)
    m1 = jnp.max(g, axis=1, keepdims=True)
    i1 = jnp.min(jnp.where(g == m1, col, E), axis=1, keepdims=True)
    g2 = jnp.where(col == i1, -1.0, g)
    m2 = jnp.max(g2, axis=1, keepdims=True)
    i2 = jnp.min(jnp.where(g2 == m2, col, E), axis=1, keepdims=True)
    ti_ref[...] = jnp.concatenate([i1, i2], axis=1)
    tv_ref[...] = jnp.concatenate([m1, m2], axis=1)


def _gate(x2d, Wg, bg):
    return pl.pallas_call(
        _gate_kernel,
        out_shape=(
            jax.ShapeDtypeStruct((T, E), jnp.float32),
            jax.ShapeDtypeStruct((T, TOPK), jnp.int32),
            jax.ShapeDtypeStruct((T, TOPK), jnp.float32),
        ),
    )(x2d, Wg, bg.reshape(1, E))


def _mlp_kernel(eidc_ref, eid_ref, xs_ref, w1_ref, b1_ref, w2_ref, b2_ref,
                ys_ref):
    j = pl.program_id(0)
    e = eid_ref[j]

    @pl.when(e >= 0)
    def _():
        h = jax.lax.dot_general(xs_ref[...], w1_ref[0],
                                (((1,), (1,)), ((), ())),
                                preferred_element_type=jnp.float32)
        h = jax.nn.gelu(h + b1_ref[0], approximate=True)
        y = jax.lax.dot_general(h, w2_ref[0],
                                (((1,), (1,)), ((), ())),
                                preferred_element_type=jnp.float32)
        ys_ref[...] = y + b2_ref[0]


def _mlp(tile_eid, xs, W1, b1, W2, b2):
    def widx(j, eidc, eid):
        return (eidc[j], 0, 0)

    def bidx(j, eidc, eid):
        return (eidc[j], 0, 0)

    grid_spec = pltpu.PrefetchScalarGridSpec(
        num_scalar_prefetch=2,
        grid=(NT,),
        in_specs=[
            pl.BlockSpec((TILE, D), lambda j, eidc, eid: (j, 0)),
            pl.BlockSpec((1, DFF, D), widx),
            pl.BlockSpec((1, 1, DFF), bidx),
            pl.BlockSpec((1, D, DFF), widx),
            pl.BlockSpec((1, 1, D), bidx),
        ],
        out_specs=pl.BlockSpec((TILE, D), lambda j, eidc, eid: (j, 0)),
    )
    return pl.pallas_call(
        _mlp_kernel,
        grid_spec=grid_spec,
        out_shape=jax.ShapeDtypeStruct((R, D), jnp.float32),
    )(jnp.maximum(tile_eid, 0), tile_eid, xs,
      W1, b1.reshape(E, 1, DFF), W2, b2.reshape(E, 1, D))


def _route_jnp(ti):
    ids = jnp.concatenate([ti[:, 0], ti[:, 1]])
    cnt = jnp.zeros((E,), jnp.int32).at[ids].add(1)
    nt = (cnt + TILE - 1) // TILE
    off = TILE * (jnp.cumsum(nt) - nt)
    starts = jnp.cumsum(cnt) - cnt
    order = jnp.argsort(ids, stable=True)
    sids = ids[order]
    rank_sorted = jnp.arange(P, dtype=jnp.int32) - starts[sids]
    pos = jnp.zeros((P,), jnp.int32).at[order].set(off[sids] + rank_sorted)
    cum_nt = jnp.cumsum(nt)
    tj = jnp.arange(NT, dtype=jnp.int32)
    eid = jnp.searchsorted(cum_nt, tj, side="right").astype(jnp.int32)
    tile_eid = jnp.where(tj < cum_nt[-1], eid, -1).astype(jnp.int32)
    return pos, tile_eid


def kernel(x, W1, b1, W2, b2, Wg, bg):
    Bx, Tx, Dx = x.shape
    x2d = x.reshape(T, D)
    gate_scores, ti, tv = _gate(x2d, Wg, bg)
    pos, tile_eid = _route_jnp(ti)
    tok = jnp.concatenate([jnp.arange(T, dtype=jnp.int32)] * TOPK)
    xs = jnp.zeros((R, D), jnp.float32).at[pos].set(x2d[tok])
    ys = _mlp(tile_eid, xs, W1, b1, W2, b2)
    out = tv[:, 0:1] * ys[pos[:T]] + tv[:, 1:2] * ys[pos[T:]]
    return out.reshape(Bx, Tx, Dx), gate_scores.reshape(Bx, Tx, E)


# speedup vs baseline: 7.6034x; 1.3655x over previous
import functools

import jax
import jax.numpy as jnp
from jax import lax
from jax.experimental import pallas as pl
from jax.experimental.pallas import tpu as pltpu
from jax.experimental.pallas import tpu_sc as plsc

E = 64
TOPK = 2
T = 2048
D = 768
DFF = 4 * D
P = T * TOPK
TILE = 128
NT = 96
R = NT * TILE

NC = 2
NS = 16
CP = P // NS
DP = CP // NC
NV = CP // 16
CT = T // (NC * NS)


def _gate_kernel(x_ref, wg_ref, bg_ref, gs_ref, ti_ref, tv_ref):
    g = jax.lax.dot_general(x_ref[...], wg_ref[...],
                            (((1,), (1,)), ((), ())),
                            preferred_element_type=jnp.float32)
    g = jax.nn.sigmoid(g + bg_ref[...])
    gs_ref[...] = g
    col = jax.lax.broadcasted_iota(jnp.int32, (T, E), 1)
    m1 = jnp.max(g, axis=1, keepdims=True)
    i1 = jnp.min(jnp.where(g == m1, col, E), axis=1, keepdims=True)
    g2 = jnp.where(col == i1, -1.0, g)
    m2 = jnp.max(g2, axis=1, keepdims=True)
    i2 = jnp.min(jnp.where(g2 == m2, col, E), axis=1, keepdims=True)
    ti_ref[...] = jnp.concatenate([i1, i2], axis=1)
    tv_ref[...] = jnp.concatenate([m1, m2], axis=1)


def _gate(x2d, Wg, bg):
    return pl.pallas_call(
        _gate_kernel,
        out_shape=(
            jax.ShapeDtypeStruct((T, E), jnp.float32),
            jax.ShapeDtypeStruct((T, TOPK), jnp.int32),
            jax.ShapeDtypeStruct((T, TOPK), jnp.float32),
        ),
    )(x2d, Wg, bg.reshape(1, E))


def _mlp_kernel(eidc_ref, eid_ref, xs_ref, w1_ref, b1_ref, w2_ref, b2_ref,
                ys_ref):
    j = pl.program_id(0)
    e = eid_ref[j]

    @pl.when(e >= 0)
    def _():
        h = jax.lax.dot_general(xs_ref[...], w1_ref[0],
                                (((1,), (1,)), ((), ())),
                                preferred_element_type=jnp.float32)
        h = jax.nn.gelu(h + b1_ref[0], approximate=True)
        y = jax.lax.dot_general(h, w2_ref[0],
                                (((1,), (1,)), ((), ())),
                                preferred_element_type=jnp.float32)
        ys_ref[...] = y + b2_ref[0]


def _mlp(tile_eid, xs, W1, b1, W2, b2):
    def widx(j, eidc, eid):
        return (eidc[j], 0, 0)

    def bidx(j, eidc, eid):
        return (eidc[j], 0, 0)

    grid_spec = pltpu.PrefetchScalarGridSpec(
        num_scalar_prefetch=2,
        grid=(NT,),
        in_specs=[
            pl.BlockSpec((TILE, D), lambda j, eidc, eid: (j, 0)),
            pl.BlockSpec((1, DFF, D), widx),
            pl.BlockSpec((1, 1, DFF), bidx),
            pl.BlockSpec((1, D, DFF), widx),
            pl.BlockSpec((1, 1, D), bidx),
        ],
        out_specs=pl.BlockSpec((TILE, D), lambda j, eidc, eid: (j, 0)),
    )
    return pl.pallas_call(
        _mlp_kernel,
        grid_spec=grid_spec,
        out_shape=jax.ShapeDtypeStruct((R, D), jnp.float32),
    )(jnp.maximum(tile_eid, 0), tile_eid, xs,
      W1, b1.reshape(E, 1, DFF), W2, b2.reshape(E, 1, D))


def _iota16():
    return lax.broadcasted_iota(jnp.int32, (16,), 0)


def _splat(e):
    return jnp.broadcast_to(e, (16,)).astype(jnp.int32)


def _sumsplat(v):
    incl = plsc.cumsum(v)
    rincl = lax.rev(plsc.cumsum(lax.rev(v, (0,))), (0,))
    return incl + rincl - v


def _take16(v, idx):
    dnums = jax.lax.GatherDimensionNumbers(
        offset_dims=(), collapsed_slice_dims=(0,), start_index_map=(0,))
    return jax.lax.gather(
        v, idx[:, None], dnums, (1,),
        mode=jax.lax.GatherScatterMode.PROMISE_IN_BOUNDS)


def _m2i(m):
    return jnp.where(m, jnp.full((16,), 1, jnp.int32),
                     jnp.zeros((16,), jnp.int32))


def _route_body(ids_hbm, x_hbm, xs_hbm, pos_hbm, te_hbm,
                ids_v, lc_v, shared, lcall_v, soff_v, stt_v, te_v,
                posc_v, posd_v, tok_v, tmp_v, rows_v, sem, sem2):
    c = lax.axis_index("c")
    s = lax.axis_index("s")
    iota = _iota16()
    zeros = jnp.zeros((16,), jnp.int32)

    pltpu.sync_copy(ids_hbm.at[pl.ds(s * CP, CP)], ids_v)
    idv = [ids_v[pl.ds(16 * v, 16)] for v in range(NV)]

    runs = []
    for v in range(NV):
        sk, sv = plsc.sort_key_val(idv[v], iota)
        prev = _take16(sk, jnp.maximum(iota - 1, 0))
        started = (sk != prev) | (iota == 0)
        start_idx = plsc.cummax(jnp.where(started, iota, zeros))
        rank = iota - start_idx
        nxt = _take16(sk, jnp.minimum(iota + 1, 15))
        is_last = (sk != nxt) | (iota == 15)
        runs.append((sk, sv, rank, is_last))

    for g in range(4):
        lc_v[pl.ds(16 * g, 16)] = zeros
    for sk, sv, rank, is_last in runs:
        plsc.addupdate_scatter(lc_v, [sk], rank + 1, mask=is_last)
    pltpu.sync_copy(lc_v, shared.at[pl.ds(s * E, E)])
    plsc.subcore_barrier()
    pltpu.sync_copy(shared, lcall_v)

    cnts = [zeros] * 4
    pres = [zeros] * 4
    for sp in range(NS):
        keepv = _m2i(_splat(sp) < _splat(s))
        for g in range(4):
            vv = lcall_v[pl.ds(sp * E + 16 * g, 16)]
            cnts[g] = cnts[g] + vv
            pres[g] = pres[g] + vv * keepv
    carryv = zeros
    nts = []
    for g in range(4):
        nt = jax.lax.shift_right_logical(cnts[g] + (TILE - 1), 7)
        nts.append(nt)
        incl = plsc.cumsum(nt)
        offt = incl - nt + carryv
        stt_v[pl.ds(16 * g, 16)] = offt
        soff_v[pl.ds(16 * g, 16)] = offt * TILE + pres[g]
        carryv = carryv + _sumsplat(nt)
    tot_tiles_v = carryv

    for v in range(NV):
        sk, sv, rank, is_last = runs[v]
        c0s = plsc.load_gather(soff_v, [sk])
        plsc.store_scatter(tmp_v, [_splat(v), sv], c0s + rank)
        posc_v[pl.ds(16 * v, 16)] = tmp_v[v, pl.ds(0, 16)]
        plsc.store_scatter(soff_v, [sk], c0s + rank + 1, mask=is_last)

    @pl.when(c == 0)
    def _():
        pltpu.sync_copy(posc_v, pos_hbm.at[pl.ds(s * CP, CP)])

    @pl.when((c == 0) & (s == 0))
    def _():
        for gt in range(NT // 16):
            te_v[pl.ds(16 * gt, 16)] = zeros
        for g in range(4):
            stg = stt_v[pl.ds(16 * g, 16)]
            evals = iota + (16 * g + 1)
            plsc.store_scatter(te_v, [stg], evals, mask=nts[g] > 0)
        carry = zeros
        for gt in range(NT // 16):
            tj = iota + 16 * gt
            cm = plsc.cummax(jnp.maximum(te_v[pl.ds(16 * gt, 16)], carry))
            carry = _take16(cm, _splat(15))
            te_v[pl.ds(16 * gt, 16)] = jnp.where(tj < tot_tiles_v, cm - 1, -1)
        pltpu.sync_copy(te_v, te_hbm)

    pbase = s * CP + c * DP
    for k in range(2):
        for j in range(4):
            off = 64 * k + 16 * j
            pv = _splat(pbase + off) + iota
            tok_v[k, pl.ds(16 * j, 16)] = jnp.where(pv >= _splat(T), pv - T,
                                                    pv)
            posd_v[k, pl.ds(16 * j, 16)] = posc_v[pl.ds(c * DP + off, 16)]
    for k in range(2):
        pltpu.async_copy(x_hbm.at[tok_v.at[k]], rows_v, sem).wait()
        pltpu.async_copy(rows_v, xs_hbm.at[posd_v.at[k]], sem2).wait()


_route = functools.partial(
    pl.kernel,
    out_type=(
        jax.ShapeDtypeStruct((R, D), jnp.float32),
        jax.ShapeDtypeStruct((P,), jnp.int32),
        jax.ShapeDtypeStruct((NT,), jnp.int32),
    ),
    mesh=plsc.VectorSubcoreMesh(core_axis_name="c", subcore_axis_name="s"),
    scratch_types=[
        pltpu.VMEM((CP,), jnp.int32),
        pltpu.VMEM((E,), jnp.int32),
        pltpu.VMEM_SHARED((NS * E,), jnp.int32),
        pltpu.VMEM((NS * E,), jnp.int32),
        pltpu.VMEM((E,), jnp.int32),
        pltpu.VMEM((E,), jnp.int32),
        pltpu.VMEM((NT,), jnp.int32),
        pltpu.VMEM((CP,), jnp.int32),
        pltpu.VMEM((2, DP // 2), jnp.int32),
        pltpu.VMEM((2, DP // 2), jnp.int32),
        pltpu.VMEM((NV, 16), jnp.int32),
        pltpu.VMEM((DP // 2, D), jnp.float32),
        pltpu.SemaphoreType.DMA,
        pltpu.SemaphoreType.DMA,
    ],
    compiler_params=pltpu.CompilerParams(needs_layout_passes=False),
)(_route_body)


def _combine_body(ys_hbm, pos_hbm, tvf_hbm, out_hbm,
                  p1_v, p2_v, tv_v, a_v, b_v, o_v, sem, sem2):
    c = lax.axis_index("c")
    s = lax.axis_index("s")
    w = s * NC + c
    tbase = w * CT
    hc = CT // 2
    for h in range(2):
        pltpu.sync_copy(pos_hbm.at[pl.ds(tbase + hc * h, hc)], p1_v.at[h])
        pltpu.sync_copy(pos_hbm.at[pl.ds(T + tbase + hc * h, hc)], p2_v.at[h])
    pltpu.sync_copy(tvf_hbm.at[pl.ds(TOPK * tbase, TOPK * CT)], tv_v)
    tvv = [tv_v[pl.ds(16 * k, 16)] for k in range(TOPK * CT // 16)]
    for h in range(2):
        pltpu.async_copy(ys_hbm.at[p1_v.at[h]], a_v, sem).wait()
        pltpu.async_copy(ys_hbm.at[p2_v.at[h]], b_v, sem2).wait()
        for i in range(hc):
            j1 = TOPK * (hc * h + i)
            v1 = _take16(tvv[j1 // 16], _splat(j1 % 16))
            v2 = _take16(tvv[(j1 + 1) // 16], _splat((j1 + 1) % 16))

            def colbody(j, _, i=i, v1=v1, v2=v2):
                o_v[i, pl.ds(16 * j, 16)] = (
                    v1 * a_v[i, pl.ds(16 * j, 16)]
                    + v2 * b_v[i, pl.ds(16 * j, 16)])
                return 0

            lax.fori_loop(0, D // 16, colbody, 0)
        pltpu.sync_copy(o_v, out_hbm.at[pl.ds(tbase + hc * h, hc)])


_combine = functools.partial(
    pl.kernel,
    out_type=jax.ShapeDtypeStruct((T, D), jnp.float32),
    mesh=plsc.VectorSubcoreMesh(core_axis_name="c", subcore_axis_name="s"),
    scratch_types=[
        pltpu.VMEM((2, CT // 2), jnp.int32),
        pltpu.VMEM((2, CT // 2), jnp.int32),
        pltpu.VMEM((TOPK * CT,), jnp.float32),
        pltpu.VMEM((CT // 2, D), jnp.float32),
        pltpu.VMEM((CT // 2, D), jnp.float32),
        pltpu.VMEM((CT // 2, D), jnp.float32),
        pltpu.SemaphoreType.DMA,
        pltpu.SemaphoreType.DMA,
    ],
    compiler_params=pltpu.CompilerParams(needs_layout_passes=False),
)(_combine_body)


def kernel(x, W1, b1, W2, b2, Wg, bg):
    Bx, Tx, Dx = x.shape
    x2d = x.reshape(T, D)
    gate_scores, ti, tv = _gate(x2d, Wg, bg)
    ids = jnp.concatenate([ti[:, 0], ti[:, 1]])
    xs, pos, tile_eid = _route(ids, x2d)
    ys = _mlp(tile_eid, xs, W1, b1, W2, b2)
    out = _combine(ys, pos, tv.reshape(P))
    return out.reshape(Bx, Tx, Dx), gate_scores.reshape(Bx, Tx, E)


# speedup vs baseline: 7.6091x; 1.0008x over previous
import functools

import jax
import jax.numpy as jnp
from jax import lax
from jax.experimental import pallas as pl
from jax.experimental.pallas import tpu as pltpu
from jax.experimental.pallas import tpu_sc as plsc

E = 64
TOPK = 2
T = 2048
D = 768
DFF = 4 * D
P = T * TOPK
TILE = 128
NT = 96
R = NT * TILE

NC = 2
NS = 16
CP = P // NS
DP = CP // NC
NV = CP // 16
CT = T // (NC * NS)


def _gate_kernel(x_ref, wg_ref, bg_ref, gs_ref, ti_ref, tv_ref):
    g = jax.lax.dot_general(x_ref[...], wg_ref[...],
                            (((1,), (1,)), ((), ())),
                            preferred_element_type=jnp.float32)
    g = jax.nn.sigmoid(g + bg_ref[...])
    gs_ref[...] = g
    col = jax.lax.broadcasted_iota(jnp.int32, (T, E), 1)
    m1 = jnp.max(g, axis=1, keepdims=True)
    i1 = jnp.min(jnp.where(g == m1, col, E), axis=1, keepdims=True)
    g2 = jnp.where(col == i1, -1.0, g)
    m2 = jnp.max(g2, axis=1, keepdims=True)
    i2 = jnp.min(jnp.where(g2 == m2, col, E), axis=1, keepdims=True)
    ti_ref[...] = jnp.concatenate([i1, i2], axis=1)
    tv_ref[...] = jnp.concatenate([m1, m2], axis=1)


def _gate(x2d, Wg, bg):
    return pl.pallas_call(
        _gate_kernel,
        out_shape=(
            jax.ShapeDtypeStruct((T, E), jnp.float32),
            jax.ShapeDtypeStruct((T, TOPK), jnp.int32),
            jax.ShapeDtypeStruct((T, TOPK), jnp.float32),
        ),
    )(x2d, Wg, bg.reshape(1, E))


def _mlp_kernel(eidc_ref, eid_ref, xs_ref, w1_ref, b1_ref, w2_ref, b2_ref,
                ys_ref):
    j = pl.program_id(0)
    e = eid_ref[j]

    @pl.when(e >= 0)
    def _():
        h = jax.lax.dot_general(xs_ref[...], w1_ref[0],
                                (((1,), (1,)), ((), ())),
                                preferred_element_type=jnp.float32)
        h = jax.nn.gelu(h + b1_ref[0], approximate=True)
        y = jax.lax.dot_general(h, w2_ref[0],
                                (((1,), (1,)), ((), ())),
                                preferred_element_type=jnp.float32)
        ys_ref[...] = y + b2_ref[0]


def _mlp(tile_eid, xs, W1, b1, W2, b2):
    def widx(j, eidc, eid):
        return (eidc[j], 0, 0)

    def bidx(j, eidc, eid):
        return (eidc[j], 0, 0)

    grid_spec = pltpu.PrefetchScalarGridSpec(
        num_scalar_prefetch=2,
        grid=(NT,),
        in_specs=[
            pl.BlockSpec((TILE, D), lambda j, eidc, eid: (j, 0)),
            pl.BlockSpec((1, DFF, D), widx),
            pl.BlockSpec((1, 1, DFF), bidx),
            pl.BlockSpec((1, D, DFF), widx),
            pl.BlockSpec((1, 1, D), bidx),
        ],
        out_specs=pl.BlockSpec((TILE, D), lambda j, eidc, eid: (j, 0)),
    )
    return pl.pallas_call(
        _mlp_kernel,
        grid_spec=grid_spec,
        out_shape=jax.ShapeDtypeStruct((R, D), jnp.float32),
    )(jnp.maximum(tile_eid, 0), tile_eid, xs,
      W1, b1.reshape(E, 1, DFF), W2, b2.reshape(E, 1, D))


def _iota16():
    return lax.broadcasted_iota(jnp.int32, (16,), 0)


def _splat(e):
    return jnp.broadcast_to(e, (16,)).astype(jnp.int32)


def _sumsplat(v):
    incl = plsc.cumsum(v)
    rincl = lax.rev(plsc.cumsum(lax.rev(v, (0,))), (0,))
    return incl + rincl - v


def _take16(v, idx):
    dnums = jax.lax.GatherDimensionNumbers(
        offset_dims=(), collapsed_slice_dims=(0,), start_index_map=(0,))
    return jax.lax.gather(
        v, idx[:, None], dnums, (1,),
        mode=jax.lax.GatherScatterMode.PROMISE_IN_BOUNDS)


def _m2i(m):
    return jnp.where(m, jnp.full((16,), 1, jnp.int32),
                     jnp.zeros((16,), jnp.int32))


def _route_body(ids_hbm, x_hbm, xs_hbm, pos_hbm, te_hbm,
                ids_v, lc_v, shared, lcall_v, soff_v, stt_v, te_v,
                posc_v, posd_v, tok_v, tmp_v, rows_v, sem, sem2):
    c = lax.axis_index("c")
    s = lax.axis_index("s")
    iota = _iota16()
    zeros = jnp.zeros((16,), jnp.int32)

    pltpu.sync_copy(ids_hbm.at[pl.ds(s * CP, CP)], ids_v)
    idv = [ids_v[pl.ds(16 * v, 16)] for v in range(NV)]

    runs = []
    for v in range(NV):
        sk, sv = plsc.sort_key_val(idv[v], iota)
        prev = _take16(sk, jnp.maximum(iota - 1, 0))
        started = (sk != prev) | (iota == 0)
        start_idx = plsc.cummax(jnp.where(started, iota, zeros))
        rank = iota - start_idx
        nxt = _take16(sk, jnp.minimum(iota + 1, 15))
        is_last = (sk != nxt) | (iota == 15)
        runs.append((sk, sv, rank, is_last))

    for g in range(4):
        lc_v[pl.ds(16 * g, 16)] = zeros
    for sk, sv, rank, is_last in runs:
        plsc.addupdate_scatter(lc_v, [sk], rank + 1, mask=is_last)
    pltpu.sync_copy(lc_v, shared.at[pl.ds(s * E, E)])
    plsc.subcore_barrier()
    pltpu.sync_copy(shared, lcall_v)

    cnts = [zeros] * 4
    pres = [zeros] * 4
    for sp in range(NS):
        keepv = _m2i(_splat(sp) < _splat(s))
        for g in range(4):
            vv = lcall_v[pl.ds(sp * E + 16 * g, 16)]
            cnts[g] = cnts[g] + vv
            pres[g] = pres[g] + vv * keepv
    carryv = zeros
    nts = []
    for g in range(4):
        nt = jax.lax.shift_right_logical(cnts[g] + (TILE - 1), 7)
        nts.append(nt)
        incl = plsc.cumsum(nt)
        offt = incl - nt + carryv
        stt_v[pl.ds(16 * g, 16)] = offt
        soff_v[pl.ds(16 * g, 16)] = offt * TILE + pres[g]
        carryv = carryv + _sumsplat(nt)
    tot_tiles_v = carryv

    for v in range(NV):
        sk, sv, rank, is_last = runs[v]
        c0s = plsc.load_gather(soff_v, [sk])
        plsc.store_scatter(tmp_v, [_splat(v), sv], c0s + rank)
        posc_v[pl.ds(16 * v, 16)] = tmp_v[v, pl.ds(0, 16)]
        plsc.store_scatter(soff_v, [sk], c0s + rank + 1, mask=is_last)

    @pl.when(c == 0)
    def _():
        pltpu.sync_copy(posc_v, pos_hbm.at[pl.ds(s * CP, CP)])

    @pl.when((c == 0) & (s == 0))
    def _():
        for gt in range(NT // 16):
            te_v[pl.ds(16 * gt, 16)] = zeros
        for g in range(4):
            stg = stt_v[pl.ds(16 * g, 16)]
            evals = iota + (16 * g + 1)
            plsc.store_scatter(te_v, [stg], evals, mask=nts[g] > 0)
        carry = zeros
        for gt in range(NT // 16):
            tj = iota + 16 * gt
            cm = plsc.cummax(jnp.maximum(te_v[pl.ds(16 * gt, 16)], carry))
            carry = _take16(cm, _splat(15))
            te_v[pl.ds(16 * gt, 16)] = jnp.where(tj < tot_tiles_v, cm - 1, -1)
        pltpu.sync_copy(te_v, te_hbm)

    pbase = s * CP + c * DP
    for k in range(2):
        for j in range(4):
            off = 64 * k + 16 * j
            pv = _splat(pbase + off) + iota
            tok_v[k, pl.ds(16 * j, 16)] = jnp.where(pv >= _splat(T), pv - T,
                                                    pv)
            posd_v[k, pl.ds(16 * j, 16)] = posc_v[pl.ds(c * DP + off, 16)]
    for k in range(2):
        pltpu.async_copy(x_hbm.at[tok_v.at[k]], rows_v, sem).wait()
        pltpu.async_copy(rows_v, xs_hbm.at[posd_v.at[k]], sem2).wait()


_route = functools.partial(
    pl.kernel,
    out_type=(
        jax.ShapeDtypeStruct((R, D), jnp.float32),
        jax.ShapeDtypeStruct((P,), jnp.int32),
        jax.ShapeDtypeStruct((NT,), jnp.int32),
    ),
    mesh=plsc.VectorSubcoreMesh(core_axis_name="c", subcore_axis_name="s"),
    scratch_types=[
        pltpu.VMEM((CP,), jnp.int32),
        pltpu.VMEM((E,), jnp.int32),
        pltpu.VMEM_SHARED((NS * E,), jnp.int32),
        pltpu.VMEM((NS * E,), jnp.int32),
        pltpu.VMEM((E,), jnp.int32),
        pltpu.VMEM((E,), jnp.int32),
        pltpu.VMEM((NT,), jnp.int32),
        pltpu.VMEM((CP,), jnp.int32),
        pltpu.VMEM((2, DP // 2), jnp.int32),
        pltpu.VMEM((2, DP // 2), jnp.int32),
        pltpu.VMEM((NV, 16), jnp.int32),
        pltpu.VMEM((DP // 2, D), jnp.float32),
        pltpu.SemaphoreType.DMA,
        pltpu.SemaphoreType.DMA,
    ],
    compiler_params=pltpu.CompilerParams(needs_layout_passes=False),
)(_route_body)


def _combine_body(ys_hbm, pos_hbm, tvf_hbm, out_hbm,
                  p1_v, p2_v, tv_v, a_v, b_v, o_v, sem, sem2):
    c = lax.axis_index("c")
    s = lax.axis_index("s")
    w = s * NC + c
    tbase = w * CT
    hc = CT // 2
    for h in range(2):
        pltpu.sync_copy(pos_hbm.at[pl.ds(tbase + hc * h, hc)], p1_v.at[h])
        pltpu.sync_copy(pos_hbm.at[pl.ds(T + tbase + hc * h, hc)], p2_v.at[h])
    pltpu.sync_copy(tvf_hbm.at[pl.ds(TOPK * tbase, TOPK * CT)], tv_v)
    tvv = [tv_v[pl.ds(16 * k, 16)] for k in range(TOPK * CT // 16)]
    for h in range(2):
        pltpu.async_copy(ys_hbm.at[p1_v.at[h]], a_v, sem).wait()
        pltpu.async_copy(ys_hbm.at[p2_v.at[h]], b_v, sem2).wait()
        for i in range(hc):
            j1 = TOPK * (hc * h + i)
            v1 = _take16(tvv[j1 // 16], _splat(j1 % 16))
            v2 = _take16(tvv[(j1 + 1) // 16], _splat((j1 + 1) % 16))

            def colbody(j, _, i=i, v1=v1, v2=v2):
                for u in range(4):
                    o_v[i, pl.ds(64 * j + 16 * u, 16)] = (
                        v1 * a_v[i, pl.ds(64 * j + 16 * u, 16)]
                        + v2 * b_v[i, pl.ds(64 * j + 16 * u, 16)])
                return 0

            lax.fori_loop(0, D // 64, colbody, 0)
        pltpu.sync_copy(o_v, out_hbm.at[pl.ds(tbase + hc * h, hc)])


_combine = functools.partial(
    pl.kernel,
    out_type=jax.ShapeDtypeStruct((T, D), jnp.float32),
    mesh=plsc.VectorSubcoreMesh(core_axis_name="c", subcore_axis_name="s"),
    scratch_types=[
        pltpu.VMEM((2, CT // 2), jnp.int32),
        pltpu.VMEM((2, CT // 2), jnp.int32),
        pltpu.VMEM((TOPK * CT,), jnp.float32),
        pltpu.VMEM((CT // 2, D), jnp.float32),
        pltpu.VMEM((CT // 2, D), jnp.float32),
        pltpu.VMEM((CT // 2, D), jnp.float32),
        pltpu.SemaphoreType.DMA,
        pltpu.SemaphoreType.DMA,
    ],
    compiler_params=pltpu.CompilerParams(needs_layout_passes=False),
)(_combine_body)


def kernel(x, W1, b1, W2, b2, Wg, bg):
    Bx, Tx, Dx = x.shape
    x2d = x.reshape(T, D)
    gate_scores, ti, tv = _gate(x2d, Wg, bg)
    ids = jnp.concatenate([ti[:, 0], ti[:, 1]])
    xs, pos, tile_eid = _route(ids, x2d)
    ys = _mlp(tile_eid, xs, W1, b1, W2, b2)
    out = _combine(ys, pos, tv.reshape(P))
    return out.reshape(Bx, Tx, Dx), gate_scores.reshape(Bx, Tx, E)
